# baseline (speedup 1.0000x reference)
import functools

import jax
import jax.numpy as jnp
from jax.experimental import pallas as pl
from jax.experimental.pallas import tpu as pltpu

LANE = 128


def _round_up(x, m):
    return (x + m - 1) // m * m


def _conv_bn_stats_kernel(x_ref, pscale_ref, pshift_ref, w_ref, b_ref,
                          y_ref, s_ref, ss_ref,
                          xp_ref, b3_ref, *, apply_prologue):
    H, W, Cout = y_ref.shape[1], y_ref.shape[2], y_ref.shape[3]
    Cin = x_ref.shape[3]
    WP = _round_up(W + 2, 8)
    F = (H + 4) * WP
    M = H * WP

    x = x_ref[0]
    if apply_prologue:
        x = jnp.maximum(x * pscale_ref[...] + pshift_ref[...], 0.0)

    xp_ref[0:2 * WP, :] = jnp.zeros((2 * WP, Cin), jnp.float32)
    xp_ref[(H + 2) * WP:F, :] = jnp.zeros((2 * WP, Cin), jnp.float32)
    for i in range(H):
        base = (i + 2) * WP
        xp_ref[base:base + W, :] = x[i]
        xp_ref[base + W:base + WP, :] = jnp.zeros((WP - W, Cin), jnp.float32)

    b3_ref[1:F, 0:Cin] = xp_ref[0:F - 1, :]
    b3_ref[0:F, Cin:2 * Cin] = xp_ref[0:F, :]
    b3_ref[0:F - 1, 2 * Cin:3 * Cin] = xp_ref[1:F, :]

    y = jnp.dot(b3_ref[WP:WP + M, :], w_ref[0],
                preferred_element_type=jnp.float32)
    y = y + jnp.dot(b3_ref[2 * WP:2 * WP + M, :], w_ref[1],
                    preferred_element_type=jnp.float32)
    y = y + jnp.dot(b3_ref[3 * WP:3 * WP + M, :], w_ref[2],
                    preferred_element_type=jnp.float32)
    y = y + b_ref[...]

    yv = y.reshape(H, WP, Cout)[:, 0:W, :]
    y_ref[0] = yv
    yf = yv.reshape(H * W, Cout)
    s_ref[0] = jnp.sum(yf, axis=0, keepdims=True)
    ss_ref[0] = jnp.sum(yf * yf, axis=0, keepdims=True)


def _conv3x3_bn_stats(x, w_mat, b, pre_scale, pre_shift, *, apply_prologue):
    N, H, W, Cin = x.shape
    Cout = w_mat.shape[2]
    WP = _round_up(W + 2, 8)
    _body = functools.partial(_conv_bn_stats_kernel, apply_prologue=apply_prologue)
    flops = 2 * N * H * WP * 9 * Cin * Cout
    bytes_accessed = 4 * (x.size + w_mat.size + N * H * W * Cout)
    return pl.pallas_call(
        _body,
        out_shape=(jax.ShapeDtypeStruct((N, H, W, Cout), jnp.float32),
                   jax.ShapeDtypeStruct((N, 1, Cout), jnp.float32),
                   jax.ShapeDtypeStruct((N, 1, Cout), jnp.float32)),
        grid=(N,),
        in_specs=[
            pl.BlockSpec((1, H, W, Cin), lambda n: (n, 0, 0, 0)),
            pl.BlockSpec((1, Cin), lambda n: (0, 0)),
            pl.BlockSpec((1, Cin), lambda n: (0, 0)),
            pl.BlockSpec((3, 3 * Cin, Cout), lambda n: (0, 0, 0)),
            pl.BlockSpec((1, Cout), lambda n: (0, 0)),
        ],
        out_specs=(
            pl.BlockSpec((1, H, W, Cout), lambda n: (n, 0, 0, 0)),
            pl.BlockSpec((1, 1, Cout), lambda n: (n, 0, 0)),
            pl.BlockSpec((1, 1, Cout), lambda n: (n, 0, 0)),
        ),
        scratch_shapes=[
            pltpu.VMEM(((H + 4) * WP, Cin), jnp.float32),
            pltpu.VMEM(((H + 4) * WP, 3 * Cin), jnp.float32),
        ],
        compiler_params=pltpu.CompilerParams(
            dimension_semantics=("parallel",)),
        cost_estimate=pl.CostEstimate(flops=flops, transcendentals=0,
                                      bytes_accessed=bytes_accessed),
    )(x, pre_scale, pre_shift, w_mat, b)


def _bn_relu_t_kernel(y_ref, scale_ref, shift_ref, o_ref):
    H, W, C = y_ref.shape[1], y_ref.shape[2], y_ref.shape[3]
    v = jnp.maximum(
        y_ref[0].reshape(H * W, C) * scale_ref[...] + shift_ref[...], 0.0)
    o_ref[0] = v.T.reshape(C, H, W)


def _bn_relu_t(y, scale, shift):
    N, H, W, C = y.shape
    return pl.pallas_call(
        _bn_relu_t_kernel,
        out_shape=jax.ShapeDtypeStruct((N, C, H, W), jnp.float32),
        grid=(N,),
        in_specs=[pl.BlockSpec((1, H, W, C), lambda n: (n, 0, 0, 0)),
                  pl.BlockSpec((1, C), lambda n: (0, 0)),
                  pl.BlockSpec((1, C), lambda n: (0, 0))],
        out_specs=pl.BlockSpec((1, C, H, W), lambda n: (n, 0, 0, 0)),
        compiler_params=pltpu.CompilerParams(dimension_semantics=("parallel",)),
    )(y, scale, shift)


def _bn_scale_shift(s_partial, ss_partial, count, gamma, beta, eps):
    s = jnp.sum(s_partial, axis=(0, 1))
    ss = jnp.sum(ss_partial, axis=(0, 1))
    mean = s / count
    var = jnp.maximum(ss / count - mean * mean, 0.0)
    scale = gamma * jax.lax.rsqrt(var + eps)
    shift = beta - mean * scale
    return scale.reshape(1, -1), shift.reshape(1, -1)


def _prep_w(w, ci, co, cpi, cpo):
    wp = jnp.zeros((3, 3, cpi, cpo), jnp.float32)
    wp = wp.at[:, :, :ci, :co].set(w.astype(jnp.float32))
    return wp.reshape(3, 3 * cpi, cpo)


def _pad_vec(v, cp):
    return jnp.pad(v.astype(jnp.float32), (0, cp - v.shape[0]))


def _double_conv_forward(x_nchw, params, eps=1e-5):
    N, Cin, H, W = x_nchw.shape
    Cout = params["w1"].shape[-1]
    cp_in, cp_out = _round_up(Cin, LANE), _round_up(Cout, LANE)

    w1 = _prep_w(params["w1"], Cin, Cout, cp_in, cp_out)
    w2 = _prep_w(params["w2"], Cout, Cout, cp_out, cp_out)
    b1 = _pad_vec(params["b1"], cp_out).reshape(1, cp_out)
    b2 = _pad_vec(params["b2"], cp_out).reshape(1, cp_out)
    g1, be1 = _pad_vec(params["g1"], cp_out), _pad_vec(params["be1"], cp_out)
    g2, be2 = _pad_vec(params["g2"], cp_out), _pad_vec(params["be2"], cp_out)

    x = jnp.transpose(x_nchw, (0, 2, 3, 1)).astype(jnp.float32)
    if cp_in != Cin:
        x = jnp.pad(x, ((0, 0), (0, 0), (0, 0), (0, cp_in - Cin)))

    count = float(N * H * W)
    ident = jnp.ones((1, cp_in), jnp.float32)
    zeros = jnp.zeros((1, cp_in), jnp.float32)

    y1, s1, ss1 = _conv3x3_bn_stats(x, w1, b1, ident, zeros,
                                    apply_prologue=False)
    sc1, sh1 = _bn_scale_shift(s1, ss1, count, g1, be1, eps)

    y2, s2, ss2 = _conv3x3_bn_stats(y1, w2, b2, sc1, sh1,
                                    apply_prologue=True)
    sc2, sh2 = _bn_scale_shift(s2, ss2, count, g2, be2, eps)

    out = _bn_relu_t(y2[..., :cp_out], sc2, sh2)
    return out[:, :Cout]


_double_conv_forward = jax.jit(_double_conv_forward)


def kernel(x, w1, b1, g1, be1, w2, b2, g2, be2):
    params = {"w1": w1, "b1": b1, "g1": g1, "be1": be1,
              "w2": w2, "b2": b2, "g2": g2, "be2": be2}
    return _double_conv_forward(x, params)


# speedup vs baseline: 1.7311x; 1.7311x over previous
import functools

import jax
import jax.numpy as jnp
from jax.experimental import pallas as pl
from jax.experimental.pallas import tpu as pltpu

LANE = 128


def _round_up(x, m):
    return (x + m - 1) // m * m


def _conv_bn_stats_kernel(x_ref, pscale_ref, pshift_ref, w_ref, b_ref,
                          y_ref, s_ref, ss_ref,
                          xp_ref, b3_ref, *, apply_prologue):
    H, W, Cout = y_ref.shape[1], y_ref.shape[2], y_ref.shape[3]
    Cin = x_ref.shape[3]
    WP = _round_up(W + 2, 8)
    F = (H + 4) * WP
    M = H * WP

    x = x_ref[0]
    if apply_prologue:
        x = jnp.maximum(x * pscale_ref[...] + pshift_ref[...], 0.0)

    xp_ref[0:2 * WP, :] = jnp.zeros((2 * WP, Cin), jnp.float32)
    xp_ref[(H + 2) * WP:F, :] = jnp.zeros((2 * WP, Cin), jnp.float32)
    for i in range(H):
        base = (i + 2) * WP
        xp_ref[base:base + W, :] = x[i]
        xp_ref[base + W:base + WP, :] = jnp.zeros((WP - W, Cin), jnp.float32)

    b3_ref[1:F, 0:Cin] = xp_ref[0:F - 1, :]
    b3_ref[0:F, Cin:2 * Cin] = xp_ref[0:F, :]
    b3_ref[0:F - 1, 2 * Cin:3 * Cin] = xp_ref[1:F, :]

    y = jnp.dot(b3_ref[WP:WP + M, :], w_ref[0],
                preferred_element_type=jnp.float32)
    y = y + jnp.dot(b3_ref[2 * WP:2 * WP + M, :], w_ref[1],
                    preferred_element_type=jnp.float32)
    y = y + jnp.dot(b3_ref[3 * WP:3 * WP + M, :], w_ref[2],
                    preferred_element_type=jnp.float32)
    y = y + b_ref[...]

    yv = y.reshape(H, WP, Cout)[:, 0:W, :]
    y_ref[0] = yv
    yf = yv.reshape(H * W, Cout)
    s_ref[0] = jnp.sum(yf, axis=0, keepdims=True)
    ss_ref[0] = jnp.sum(yf * yf, axis=0, keepdims=True)


def _conv3x3_bn_stats(x, w_mat, b, pre_scale, pre_shift, *, apply_prologue):
    N, H, W, Cin = x.shape
    Cout = w_mat.shape[2]
    WP = _round_up(W + 2, 8)
    _body = functools.partial(_conv_bn_stats_kernel, apply_prologue=apply_prologue)
    flops = 2 * N * H * WP * 9 * Cin * Cout
    bytes_accessed = 4 * (x.size + w_mat.size + N * H * W * Cout)
    return pl.pallas_call(
        _body,
        out_shape=(jax.ShapeDtypeStruct((N, H, W, Cout), jnp.float32),
                   jax.ShapeDtypeStruct((N, 1, Cout), jnp.float32),
                   jax.ShapeDtypeStruct((N, 1, Cout), jnp.float32)),
        grid=(N,),
        in_specs=[
            pl.BlockSpec((1, H, W, Cin), lambda n: (n, 0, 0, 0)),
            pl.BlockSpec((1, Cin), lambda n: (0, 0)),
            pl.BlockSpec((1, Cin), lambda n: (0, 0)),
            pl.BlockSpec((3, 3 * Cin, Cout), lambda n: (0, 0, 0)),
            pl.BlockSpec((1, Cout), lambda n: (0, 0)),
        ],
        out_specs=(
            pl.BlockSpec((1, H, W, Cout), lambda n: (n, 0, 0, 0)),
            pl.BlockSpec((1, 1, Cout), lambda n: (n, 0, 0)),
            pl.BlockSpec((1, 1, Cout), lambda n: (n, 0, 0)),
        ),
        scratch_shapes=[
            pltpu.VMEM(((H + 4) * WP, Cin), jnp.float32),
            pltpu.VMEM(((H + 4) * WP, 3 * Cin), jnp.float32),
        ],
        compiler_params=pltpu.CompilerParams(
            dimension_semantics=("parallel",)),
        cost_estimate=pl.CostEstimate(flops=flops, transcendentals=0,
                                      bytes_accessed=bytes_accessed),
    )(x, pre_scale, pre_shift, w_mat, b)


def _bn_scale_shift(s_partial, ss_partial, count, gamma, beta, eps):
    s = jnp.sum(s_partial, axis=(0, 1))
    ss = jnp.sum(ss_partial, axis=(0, 1))
    mean = s / count
    var = jnp.maximum(ss / count - mean * mean, 0.0)
    scale = gamma * jax.lax.rsqrt(var + eps)
    shift = beta - mean * scale
    return scale.reshape(1, -1), shift.reshape(1, -1)


def _prep_w(w, ci, co, cpi, cpo):
    wp = jnp.zeros((3, 3, cpi, cpo), jnp.float32)
    wp = wp.at[:, :, :ci, :co].set(w.astype(jnp.float32))
    return wp.reshape(3, 3 * cpi, cpo)


def _pad_vec(v, cp):
    return jnp.pad(v.astype(jnp.float32), (0, cp - v.shape[0]))


def _double_conv_forward(x_nchw, params, eps=1e-5):
    N, Cin, H, W = x_nchw.shape
    Cout = params["w1"].shape[-1]
    cp_in, cp_out = _round_up(Cin, LANE), _round_up(Cout, LANE)

    w1 = _prep_w(params["w1"], Cin, Cout, cp_in, cp_out)
    w2 = _prep_w(params["w2"], Cout, Cout, cp_out, cp_out)
    b1 = _pad_vec(params["b1"], cp_out).reshape(1, cp_out)
    b2 = _pad_vec(params["b2"], cp_out).reshape(1, cp_out)
    g1, be1 = _pad_vec(params["g1"], cp_out), _pad_vec(params["be1"], cp_out)
    g2, be2 = _pad_vec(params["g2"], cp_out), _pad_vec(params["be2"], cp_out)

    x = jnp.transpose(x_nchw, (0, 2, 3, 1)).astype(jnp.float32)
    if cp_in != Cin:
        x = jnp.pad(x, ((0, 0), (0, 0), (0, 0), (0, cp_in - Cin)))

    count = float(N * H * W)
    ident = jnp.ones((1, cp_in), jnp.float32)
    zeros = jnp.zeros((1, cp_in), jnp.float32)

    y1, s1, ss1 = _conv3x3_bn_stats(x, w1, b1, ident, zeros,
                                    apply_prologue=False)
    sc1, sh1 = _bn_scale_shift(s1, ss1, count, g1, be1, eps)

    y2, s2, ss2 = _conv3x3_bn_stats(y1, w2, b2, sc1, sh1,
                                    apply_prologue=True)
    sc2, sh2 = _bn_scale_shift(s2, ss2, count, g2, be2, eps)

    out = jnp.maximum(y2 * sc2.reshape(1, 1, 1, -1) + sh2.reshape(1, 1, 1, -1), 0.0)
    return jnp.transpose(out[..., :Cout], (0, 3, 1, 2))


_double_conv_forward = jax.jit(_double_conv_forward)


def kernel(x, w1, b1, g1, be1, w2, b2, g2, be2):
    params = {"w1": w1, "b1": b1, "g1": g1, "be1": be1,
              "w2": w2, "b2": b2, "g2": g2, "be2": be2}
    return _double_conv_forward(x, params)
